# baseline (speedup 1.0000x reference)
import jax
import jax.numpy as jnp
from jax.experimental import pallas as pl
from jax.experimental.pallas import tpu as pltpu

D_MODEL = 768
OUT_DIM = 768
NUM_EXPERTS = 8
RANK = 8
ER = NUM_EXPERTS * RANK
SCALING = 16 / 8
QUESTION_START = 611
SEQ = 2048
HALF = SEQ // 2
PAIR = 2
ALIGNED = (QUESTION_START // 8) * 8
N_QUESTION = (SEQ - 1) - QUESTION_START


def _moe_kernel(x_ref, w_ref, wr_ref, br_ref, aall_ref, bmt_ref, out_ref,
                meff_scr):
    s = pl.program_id(1)

    @pl.when(s == 0)
    def _prelude():
        sums = []
        for j in range(PAIR):
            xj = x_ref[j]
            qs = jnp.sum(xj[ALIGNED:SEQ], axis=0, keepdims=True)
            qs = qs - xj[ALIGNED:ALIGNED + 1] - xj[ALIGNED + 1:ALIGNED + 2] \
                - xj[ALIGNED + 2:ALIGNED + 3] - xj[SEQ - 1:SEQ]
            sums.append(qs)
        xagg = jnp.concatenate(sums, axis=0) * (1.0 / N_QUESTION)

        logits = jax.lax.dot_general(
            xagg, wr_ref[...], (((1,), (1,)), ((), ())),
            preferred_element_type=jnp.float32) + br_ref[...]
        mx = jnp.max(logits, axis=-1, keepdims=True)
        ex = jnp.exp(logits - mx)
        routing = ex / jnp.sum(ex, axis=-1, keepdims=True)

        rws = jax.lax.broadcasted_iota(jnp.int32, (NUM_EXPERTS, ER), 0)
        cls = jax.lax.broadcasted_iota(jnp.int32, (NUM_EXPERTS, ER), 1)
        sel = (cls // RANK == rws).astype(jnp.float32)
        w64 = jax.lax.dot_general(routing, sel, (((1,), (0,)), ((), ())),
                                  preferred_element_type=jnp.float32) * SCALING

        for j in range(PAIR):
            bw = bmt_ref[...] * w64[j:j + 1]
            meff = w_ref[...] + jax.lax.dot_general(
                bw, aall_ref[...], (((1,), (0,)), ((), ())),
                preferred_element_type=jnp.float32)
            meff_scr[j] = meff.astype(jnp.bfloat16)

    for j in range(PAIR):
        xh = x_ref[j, pl.ds(s * HALF, HALF), :].astype(jnp.bfloat16)
        out_ref[j] = jax.lax.dot_general(
            xh, meff_scr[j], (((1,), (1,)), ((), ())),
            preferred_element_type=jnp.float32)


@jax.jit
def kernel(x, W, b, Wr, br, A, Bm):
    B, S, D = x.shape
    aall = A.reshape(ER, D)
    bmt = jnp.transpose(Bm, (1, 0, 2)).reshape(OUT_DIM, ER)
    br2 = br.reshape(1, NUM_EXPERTS)

    return pl.pallas_call(
        _moe_kernel,
        grid=(B // PAIR, 2),
        in_specs=[
            pl.BlockSpec((PAIR, S, D), lambda i, s: (i, 0, 0)),
            pl.BlockSpec((OUT_DIM, D), lambda i, s: (0, 0)),
            pl.BlockSpec((NUM_EXPERTS, D), lambda i, s: (0, 0)),
            pl.BlockSpec((1, NUM_EXPERTS), lambda i, s: (0, 0)),
            pl.BlockSpec((ER, D), lambda i, s: (0, 0)),
            pl.BlockSpec((OUT_DIM, ER), lambda i, s: (0, 0)),
        ],
        out_specs=pl.BlockSpec((PAIR, HALF, OUT_DIM), lambda i, s: (i, s, 0)),
        out_shape=jax.ShapeDtypeStruct((B, S, OUT_DIM), jnp.float32),
        scratch_shapes=[pltpu.VMEM((PAIR, OUT_DIM, D_MODEL), jnp.bfloat16)],
        compiler_params=pltpu.CompilerParams(
            dimension_semantics=("arbitrary", "arbitrary"),
            vmem_limit_bytes=100 * 1024 * 1024),
    )(x, W, Wr, br2, aall, bmt)
